# baseline (speedup 1.0000x reference)
import functools
import math

import jax
import jax.numpy as jnp
from jax.experimental import pallas as pl
from jax.experimental.pallas import tpu as pltpu

_K = 16
_EPS = 1e-05
_EPS_VAR = 1e-04
_SQRT_2_OVER_PI = math.sqrt(2.0 / math.pi)


def _oddeven_merge_sort_pairs(n):
    pairs = []

    def merge(lo, hi, r):
        step = r * 2
        if step < hi - lo:
            merge(lo, hi, step)
            merge(lo + r, hi, step)
            for i in range(lo + r, hi - r, step):
                pairs.append((i, i + r))
        else:
            pairs.append((lo, lo + r))

    def sort(lo, hi):
        if hi - lo >= 2:
            mid = lo + (hi - lo) // 2
            sort(lo, mid)
            sort(mid, hi)
            merge(lo, hi, 1)

    sort(0, n)
    return pairs


def _gelu(x):
    return 0.5 * x * (1.0 + jnp.tanh(_SQRT_2_OVER_PI * (x + 0.044715 * x * x * x)))


def _colsum(a):
    ones = jnp.ones((1, a.shape[0]), jnp.float32)
    return jax.lax.dot_general(ones, a, (((1,), (0,)), ((), ())),
                               preferred_element_type=jnp.float32)


def _stats_kernel(x_ref, s_ref):
    i = pl.program_id(0)
    xb = x_ref[...]
    g = _gelu(xb)
    s0 = _colsum(xb)
    s1 = _colsum(xb * xb)
    s2 = _colsum(g)
    part = jnp.concatenate([s0, s1, s2], axis=0)

    @pl.when(i == 0)
    def _():
        s_ref[...] = jnp.zeros_like(s_ref)

    s_ref[...] += part


def _main_kernel(x_ref, s_ref, p_ref, o_ref, *, n_tokens, k):
    d = x_ref.shape[1]
    inv_n = 1.0 / n_tokens
    s = s_ref[...]
    mu = s[0:1, :] * inv_n
    ex2 = s[1:2, :] * inv_n
    var = jnp.maximum(ex2 - mu * mu, _EPS_VAR)
    std = jnp.sqrt(var)
    om = s[2:3, :] * inv_n
    onorm = jnp.sqrt(jnp.sum(om * om))
    ema_n = om / jnp.maximum(onorm, 1e-12)

    tau = p_ref[0]
    beta_up = p_ref[1]
    beta_dn = p_ref[2]
    gamma = p_ref[3]

    xb = x_ref[...]
    rstd = 1.0 / (std + _EPS)
    z = (xb - mu) * rstd
    az = jnp.abs(z)

    nc = d // 128
    cols = [az[:, j * 128:(j + 1) * 128] for j in range(nc)]
    for i, j in _oddeven_merge_sort_pairs(nc):
        hi = jnp.maximum(cols[i], cols[j])
        lo = jnp.minimum(cols[i], cols[j])
        cols[i] = hi
        cols[j] = lo
    for p in range(k - 1):
        m = jnp.max(cols[0], axis=1, keepdims=True)
        msk = cols[0] >= m
        for j in range(k - 1 - p):
            cols[j] = jnp.where(msk, cols[j + 1], cols[j])
    thr = jnp.max(cols[0], axis=1, keepdims=True)

    g = _gelu(xb)
    tz = jnp.tanh(gamma * z)
    gate = 1.0 + jnp.where(tz > 0.0, beta_up, beta_dn) * tz
    gate = jnp.clip(gate, 0.05, 8.0)

    cs = jax.lax.dot_general(g, ema_n.reshape(d, 1),
                             (((1,), (0,)), ((), ())),
                             preferred_element_type=jnp.float32)
    gn2 = jax.lax.dot_general(g * g, jnp.ones((d, 1), jnp.float32),
                              (((1,), (0,)), ((), ())),
                              preferred_element_type=jnp.float32)
    gn = jnp.sqrt(gn2)
    inv_gn = 1.0 / jnp.maximum(gn, 1e-12)
    cos = jnp.clip(cs * inv_gn, -1.0, 1.0)
    gate_cos = jnp.exp(-tau * cos)

    base = g * gate_cos
    o_ref[...] = jnp.where(az >= thr, base * gate, base)


def kernel(x, logit_decay, log_tau, log_beta_up, log_beta_dn, log_gamma):
    del logit_decay
    B, T, D = x.shape
    n = B * T
    k = min(_K, D)
    xf = x.reshape(n, D)

    params = jnp.stack([
        jnp.exp(log_tau),
        jax.nn.softplus(log_beta_up),
        jax.nn.softplus(log_beta_dn),
        jax.nn.softplus(log_gamma),
    ]).astype(jnp.float32)

    tb = 256
    grid = n // tb

    stats = pl.pallas_call(
        _stats_kernel,
        grid=(grid,),
        in_specs=[pl.BlockSpec((tb, D), lambda i: (i, 0))],
        out_specs=pl.BlockSpec((3, D), lambda i: (0, 0)),
        out_shape=jax.ShapeDtypeStruct((3, D), jnp.float32),
    )(xf)

    out = pl.pallas_call(
        functools.partial(_main_kernel, n_tokens=float(n), k=k),
        grid=(grid,),
        in_specs=[
            pl.BlockSpec((tb, D), lambda i: (i, 0)),
            pl.BlockSpec((3, D), lambda i: (0, 0)),
            pl.BlockSpec(memory_space=pltpu.SMEM),
        ],
        out_specs=pl.BlockSpec((tb, D), lambda i: (i, 0)),
        out_shape=jax.ShapeDtypeStruct((n, D), jnp.float32),
    )(xf, stats, params)

    return out.reshape(B, T, D)


# speedup vs baseline: 1.0093x; 1.0093x over previous
import functools
import math

import jax
import jax.numpy as jnp
from jax.experimental import pallas as pl
from jax.experimental.pallas import tpu as pltpu

_K = 16
_EPS = 1e-05
_EPS_VAR = 1e-04
_SQRT_2_OVER_PI = math.sqrt(2.0 / math.pi)


def _oddeven_merge_sort_pairs(n):
    pairs = []

    def merge(lo, hi, r):
        step = r * 2
        if step < hi - lo:
            merge(lo, hi, step)
            merge(lo + r, hi, step)
            for i in range(lo + r, hi - r, step):
                pairs.append((i, i + r))
        else:
            pairs.append((lo, lo + r))

    def sort(lo, hi):
        if hi - lo >= 2:
            mid = lo + (hi - lo) // 2
            sort(lo, mid)
            sort(mid, hi)
            merge(lo, hi, 1)

    sort(0, n)
    return pairs


def _gelu(x):
    return 0.5 * x * (1.0 + jnp.tanh(_SQRT_2_OVER_PI * (x + 0.044715 * x * x * x)))


def _colsum(a):
    ones = jnp.ones((1, a.shape[0]), jnp.float32)
    return jax.lax.dot_general(ones, a, (((1,), (0,)), ((), ())),
                               preferred_element_type=jnp.float32)


def _stats_kernel(x_ref, s_ref):
    i = pl.program_id(0)
    xb = x_ref[...]
    g = _gelu(xb)
    s0 = _colsum(xb)
    s1 = _colsum(xb * xb)
    s2 = _colsum(g)
    part = jnp.concatenate([s0, s1, s2], axis=0)

    @pl.when(i == 0)
    def _():
        s_ref[...] = jnp.zeros_like(s_ref)

    s_ref[...] += part


def _main_kernel(x_ref, s_ref, p_ref, o_ref, *, n_tokens, k):
    d = x_ref.shape[1]
    inv_n = 1.0 / n_tokens
    s = s_ref[...]
    mu = s[0:1, :] * inv_n
    ex2 = s[1:2, :] * inv_n
    var = jnp.maximum(ex2 - mu * mu, _EPS_VAR)
    std = jnp.sqrt(var)
    om = s[2:3, :] * inv_n
    onorm = jnp.sqrt(jnp.sum(om * om))
    ema_n = om / jnp.maximum(onorm, 1e-12)

    tau = p_ref[0]
    beta_up = p_ref[1]
    beta_dn = p_ref[2]
    gamma = p_ref[3]

    xb = x_ref[...]
    rstd = 1.0 / (std + _EPS)
    z = (xb - mu) * rstd
    az = jnp.abs(z)

    nc = d // 128
    cols = [az[:, j * 128:(j + 1) * 128] for j in range(nc)]
    for i, j in _oddeven_merge_sort_pairs(nc):
        hi = jnp.maximum(cols[i], cols[j])
        lo = jnp.minimum(cols[i], cols[j])
        cols[i] = hi
        cols[j] = lo
    for p in range(k - 1):
        m = jnp.max(cols[0], axis=1, keepdims=True)
        msk = cols[0] >= m
        for j in range(k - 1 - p):
            cols[j] = jnp.where(msk, cols[j + 1], cols[j])
    thr = jnp.max(cols[0], axis=1, keepdims=True)

    g = _gelu(xb)
    tz = jnp.tanh(gamma * z)
    gate = 1.0 + jnp.where(tz > 0.0, beta_up, beta_dn) * tz
    gate = jnp.clip(gate, 0.05, 8.0)

    dn = (((1,), (1,)), ((), ()))
    cs = jax.lax.dot_general(g, ema_n, dn, preferred_element_type=jnp.float32)
    gn2 = jax.lax.dot_general(g * g, jnp.ones((1, d), jnp.float32), dn,
                              preferred_element_type=jnp.float32)
    gn = jnp.sqrt(gn2)
    inv_gn = 1.0 / jnp.maximum(gn, 1e-12)
    cos = jnp.clip(cs * inv_gn, -1.0, 1.0)
    gate_cos = jnp.exp(-tau * cos)

    base = g * gate_cos
    o_ref[...] = jnp.where(az >= thr, base * gate, base)


def kernel(x, logit_decay, log_tau, log_beta_up, log_beta_dn, log_gamma):
    del logit_decay
    B, T, D = x.shape
    n = B * T
    k = min(_K, D)
    xf = x.reshape(n, D)

    params = jnp.stack([
        jnp.exp(log_tau),
        jax.nn.softplus(log_beta_up),
        jax.nn.softplus(log_beta_dn),
        jax.nn.softplus(log_gamma),
    ]).astype(jnp.float32)

    tb = 256
    grid = n // tb

    stats = pl.pallas_call(
        _stats_kernel,
        grid=(grid,),
        in_specs=[pl.BlockSpec((tb, D), lambda i: (i, 0))],
        out_specs=pl.BlockSpec((3, D), lambda i: (0, 0)),
        out_shape=jax.ShapeDtypeStruct((3, D), jnp.float32),
    )(xf)

    out = pl.pallas_call(
        functools.partial(_main_kernel, n_tokens=float(n), k=k),
        grid=(grid,),
        in_specs=[
            pl.BlockSpec((tb, D), lambda i: (i, 0)),
            pl.BlockSpec((3, D), lambda i: (0, 0)),
            pl.BlockSpec(memory_space=pltpu.SMEM),
        ],
        out_specs=pl.BlockSpec((tb, D), lambda i: (i, 0)),
        out_shape=jax.ShapeDtypeStruct((n, D), jnp.float32),
    )(xf, stats, params)

    return out.reshape(B, T, D)


# speedup vs baseline: 1.0259x; 1.0164x over previous
import functools
import math

import jax
import jax.numpy as jnp
from jax.experimental import pallas as pl
from jax.experimental.pallas import tpu as pltpu

_K = 16
_EPS = 1e-05
_EPS_VAR = 1e-04
_SQRT_2_OVER_PI = math.sqrt(2.0 / math.pi)


def _oddeven_merge_sort_pairs(n):
    pairs = []

    def merge(lo, hi, r):
        step = r * 2
        if step < hi - lo:
            merge(lo, hi, step)
            merge(lo + r, hi, step)
            for i in range(lo + r, hi - r, step):
                pairs.append((i, i + r))
        else:
            pairs.append((lo, lo + r))

    def sort(lo, hi):
        if hi - lo >= 2:
            mid = lo + (hi - lo) // 2
            sort(lo, mid)
            sort(mid, hi)
            merge(lo, hi, 1)

    sort(0, n)
    return pairs


def _gelu(x):
    return 0.5 * x * (1.0 + jnp.tanh(_SQRT_2_OVER_PI * (x + 0.044715 * x * x * x)))


def _colsum(a):
    ones = jnp.ones((1, a.shape[0]), jnp.float32)
    return jax.lax.dot_general(ones, a, (((1,), (0,)), ((), ())),
                               preferred_element_type=jnp.float32)


def _stats_kernel(x_ref, s_ref):
    i = pl.program_id(0)
    xb = x_ref[...]
    g = _gelu(xb)
    s0 = _colsum(xb)
    s1 = _colsum(xb * xb)
    s2 = _colsum(g)
    part = jnp.concatenate([s0, s1, s2], axis=0)

    @pl.when(i == 0)
    def _():
        s_ref[...] = jnp.zeros_like(s_ref)

    s_ref[...] += part


def _main_kernel(x_ref, s_ref, p_ref, o_ref, *, n_tokens, k):
    d = x_ref.shape[1]
    inv_n = 1.0 / n_tokens
    s = s_ref[...]
    mu = s[0:1, :] * inv_n
    ex2 = s[1:2, :] * inv_n
    var = jnp.maximum(ex2 - mu * mu, _EPS_VAR)
    std = jnp.sqrt(var)
    om = s[2:3, :] * inv_n
    onorm = jnp.sqrt(jnp.sum(om * om))
    ema_n = om / jnp.maximum(onorm, 1e-12)

    tau = p_ref[0]
    beta_up = p_ref[1]
    beta_dn = p_ref[2]
    gamma = p_ref[3]

    xb = x_ref[...]
    rstd = 1.0 / (std + _EPS)
    z = (xb - mu) * rstd
    az = jnp.abs(z)

    nc = d // 128
    cols = [az[:, j * 128:(j + 1) * 128] for j in range(nc)]
    for i, j in _oddeven_merge_sort_pairs(nc):
        hi = jnp.maximum(cols[i], cols[j])
        lo = jnp.minimum(cols[i], cols[j])
        cols[i] = hi
        cols[j] = lo
    for p in range(k - 1):
        m = jnp.max(cols[0], axis=1, keepdims=True)
        msk = cols[0] >= m
        for j in range(k - 1 - p):
            cols[j] = jnp.where(msk, cols[j + 1], cols[j])
    thr = jnp.max(cols[0], axis=1, keepdims=True)

    g = _gelu(xb)
    tz = jnp.tanh(gamma * z)
    gate = 1.0 + jnp.where(tz > 0.0, beta_up, beta_dn) * tz
    gate = jnp.clip(gate, 0.05, 8.0)
    gate = jnp.where(az >= thr, gate, 1.0)

    dn = (((1,), (1,)), ((), ()))
    cs = jax.lax.dot_general(g, ema_n, dn, preferred_element_type=jnp.float32)
    gn2 = jax.lax.dot_general(g * g, jnp.ones((1, d), jnp.float32), dn,
                              preferred_element_type=jnp.float32)
    gn = jnp.sqrt(gn2)
    inv_gn = 1.0 / jnp.maximum(gn, 1e-12)
    cos = jnp.clip(cs * inv_gn, -1.0, 1.0)
    gate_cos = jnp.exp(-tau * cos)

    o_ref[...] = g * gate * gate_cos


def kernel(x, logit_decay, log_tau, log_beta_up, log_beta_dn, log_gamma):
    del logit_decay
    B, T, D = x.shape
    n = B * T
    k = min(_K, D)
    xf = x.reshape(n, D)

    params = jnp.stack([
        jnp.exp(log_tau),
        jax.nn.softplus(log_beta_up),
        jax.nn.softplus(log_beta_dn),
        jax.nn.softplus(log_gamma),
    ]).astype(jnp.float32)

    tb = 256
    grid = n // tb

    stats = pl.pallas_call(
        _stats_kernel,
        grid=(grid,),
        in_specs=[pl.BlockSpec((tb, D), lambda i: (i, 0))],
        out_specs=pl.BlockSpec((3, D), lambda i: (0, 0)),
        out_shape=jax.ShapeDtypeStruct((3, D), jnp.float32),
    )(xf)

    out = pl.pallas_call(
        functools.partial(_main_kernel, n_tokens=float(n), k=k),
        grid=(grid,),
        in_specs=[
            pl.BlockSpec((tb, D), lambda i: (i, 0)),
            pl.BlockSpec((3, D), lambda i: (0, 0)),
            pl.BlockSpec(memory_space=pltpu.SMEM),
        ],
        out_specs=pl.BlockSpec((tb, D), lambda i: (i, 0)),
        out_shape=jax.ShapeDtypeStruct((n, D), jnp.float32),
    )(xf, stats, params)

    return out.reshape(B, T, D)


# speedup vs baseline: 1.0496x; 1.0231x over previous
import functools
import math

import jax
import jax.numpy as jnp
from jax.experimental import pallas as pl
from jax.experimental.pallas import tpu as pltpu

_K = 16
_EPS = 1e-05
_EPS_VAR = 1e-04
_SQRT_2_OVER_PI = math.sqrt(2.0 / math.pi)


def _oddeven_merge_sort_pairs(n):
    pairs = []

    def merge(lo, hi, r):
        step = r * 2
        if step < hi - lo:
            merge(lo, hi, step)
            merge(lo + r, hi, step)
            for i in range(lo + r, hi - r, step):
                pairs.append((i, i + r))
        else:
            pairs.append((lo, lo + r))

    def sort(lo, hi):
        if hi - lo >= 2:
            mid = lo + (hi - lo) // 2
            sort(lo, mid)
            sort(mid, hi)
            merge(lo, hi, 1)

    sort(0, n)
    return pairs


def _gelu(x):
    return 0.5 * x * (1.0 + jnp.tanh(_SQRT_2_OVER_PI * (x + 0.044715 * x * x * x)))


def _colsum(a):
    ones = jnp.ones((1, a.shape[0]), jnp.float32)
    return jax.lax.dot_general(ones, a, (((1,), (0,)), ((), ())),
                               preferred_element_type=jnp.float32)


def _stats_kernel(x_ref, s_ref):
    i = pl.program_id(0)
    xb = x_ref[...]
    g = _gelu(xb)
    s0 = _colsum(xb)
    s1 = _colsum(xb * xb)
    s2 = _colsum(g)
    part = jnp.concatenate([s0, s1, s2], axis=0)

    @pl.when(i == 0)
    def _():
        s_ref[...] = jnp.zeros_like(s_ref)

    s_ref[...] += part


def _main_kernel(x_ref, s_ref, p_ref, o_ref, *, n_tokens, k):
    d = x_ref.shape[1]
    inv_n = 1.0 / n_tokens
    s = s_ref[...]
    mu = s[0:1, :] * inv_n
    ex2 = s[1:2, :] * inv_n
    var = jnp.maximum(ex2 - mu * mu, _EPS_VAR)
    std = jnp.sqrt(var)
    om = s[2:3, :] * inv_n
    onorm = jnp.sqrt(jnp.sum(om * om))
    ema_n = om / jnp.maximum(onorm, 1e-12)

    tau = p_ref[0]
    beta_up = p_ref[1]
    beta_dn = p_ref[2]
    gamma = p_ref[3]

    xb = x_ref[...]
    rstd = 1.0 / (std + _EPS)
    z = (xb - mu) * rstd
    az = jnp.abs(z)

    nc = d // 128
    cols = [az[:, j * 128:(j + 1) * 128] for j in range(nc)]
    for i, j in _oddeven_merge_sort_pairs(nc):
        hi = jnp.maximum(cols[i], cols[j])
        lo = jnp.minimum(cols[i], cols[j])
        cols[i] = hi
        cols[j] = lo
    for p in range(k - 1):
        m = jnp.max(cols[0], axis=1, keepdims=True)
        msk = cols[0] >= m
        for j in range(k - 1 - p):
            cols[j] = jnp.where(msk, cols[j + 1], cols[j])
    thr = jnp.max(cols[0], axis=1, keepdims=True)

    g = _gelu(xb)
    tz = jnp.tanh(gamma * z)
    gate = 1.0 + jnp.where(tz > 0.0, beta_up, beta_dn) * tz
    gate = jnp.clip(gate, 0.05, 8.0)
    gate = jnp.where(az >= thr, gate, 1.0)

    dn = (((1,), (1,)), ((), ()))
    cs = jax.lax.dot_general(g, ema_n, dn, preferred_element_type=jnp.float32)
    gn2 = jax.lax.dot_general(g * g, jnp.ones((1, d), jnp.float32), dn,
                              preferred_element_type=jnp.float32)
    gn = jnp.sqrt(gn2)
    inv_gn = 1.0 / jnp.maximum(gn, 1e-12)
    cos = jnp.clip(cs * inv_gn, -1.0, 1.0)
    gate_cos = jnp.exp(-tau * cos)

    o_ref[...] = g * gate * gate_cos


def kernel(x, logit_decay, log_tau, log_beta_up, log_beta_dn, log_gamma):
    del logit_decay
    B, T, D = x.shape
    n = B * T
    k = min(_K, D)
    xf = x.reshape(n, D)

    params = jnp.stack([
        jnp.exp(log_tau),
        jax.nn.softplus(log_beta_up),
        jax.nn.softplus(log_beta_dn),
        jax.nn.softplus(log_gamma),
    ]).astype(jnp.float32)

    tb = 512
    grid = n // tb

    stats = pl.pallas_call(
        _stats_kernel,
        grid=(grid,),
        in_specs=[pl.BlockSpec((tb, D), lambda i: (i, 0))],
        out_specs=pl.BlockSpec((3, D), lambda i: (0, 0)),
        out_shape=jax.ShapeDtypeStruct((3, D), jnp.float32),
    )(xf)

    out = pl.pallas_call(
        functools.partial(_main_kernel, n_tokens=float(n), k=k),
        grid=(grid,),
        in_specs=[
            pl.BlockSpec((tb, D), lambda i: (i, 0)),
            pl.BlockSpec((3, D), lambda i: (0, 0)),
            pl.BlockSpec(memory_space=pltpu.SMEM),
        ],
        out_specs=pl.BlockSpec((tb, D), lambda i: (i, 0)),
        out_shape=jax.ShapeDtypeStruct((n, D), jnp.float32),
    )(xf, stats, params)

    return out.reshape(B, T, D)


# speedup vs baseline: 1.0713x; 1.0207x over previous
import functools
import math

import jax
import jax.numpy as jnp
from jax.experimental import pallas as pl
from jax.experimental.pallas import tpu as pltpu

_K = 16
_EPS = 1e-05
_EPS_VAR = 1e-04
_SQRT_2_OVER_PI = math.sqrt(2.0 / math.pi)


def _oddeven_merge_sort_pairs(n):
    pairs = []

    def merge(lo, hi, r):
        step = r * 2
        if step < hi - lo:
            merge(lo, hi, step)
            merge(lo + r, hi, step)
            for i in range(lo + r, hi - r, step):
                pairs.append((i, i + r))
        else:
            pairs.append((lo, lo + r))

    def sort(lo, hi):
        if hi - lo >= 2:
            mid = lo + (hi - lo) // 2
            sort(lo, mid)
            sort(mid, hi)
            merge(lo, hi, 1)

    sort(0, n)
    return pairs


def _gelu(x):
    return 0.5 * x * (1.0 + jnp.tanh(_SQRT_2_OVER_PI * (x + 0.044715 * x * x * x)))


def _colsum(a):
    ones = jnp.ones((1, a.shape[0]), jnp.float32)
    return jax.lax.dot_general(ones, a, (((1,), (0,)), ((), ())),
                               preferred_element_type=jnp.float32)


def _stats_kernel(x_ref, s_ref, g_ref):
    i = pl.program_id(0)
    xb = x_ref[...]
    g = _gelu(xb)
    g_ref[...] = g
    s0 = _colsum(xb)
    s1 = _colsum(xb * xb)
    s2 = _colsum(g)
    part = jnp.concatenate([s0, s1, s2], axis=0)

    @pl.when(i == 0)
    def _():
        s_ref[...] = jnp.zeros_like(s_ref)

    s_ref[...] += part


def _main_kernel(x_ref, g_ref, s_ref, p_ref, o_ref, *, n_tokens, k):
    d = x_ref.shape[1]
    inv_n = 1.0 / n_tokens
    s = s_ref[...]
    mu = s[0:1, :] * inv_n
    ex2 = s[1:2, :] * inv_n
    var = jnp.maximum(ex2 - mu * mu, _EPS_VAR)
    std = jnp.sqrt(var)
    om = s[2:3, :] * inv_n
    onorm = jnp.sqrt(jnp.sum(om * om))
    ema_n = om / jnp.maximum(onorm, 1e-12)

    tau = p_ref[0]
    beta_up = p_ref[1]
    beta_dn = p_ref[2]
    gamma = p_ref[3]

    xb = x_ref[...]
    rstd = 1.0 / (std + _EPS)
    z = (xb - mu) * rstd
    az = jnp.abs(z)

    nc = d // 128
    cols = [az[:, j * 128:(j + 1) * 128] for j in range(nc)]
    for i, j in _oddeven_merge_sort_pairs(nc):
        hi = jnp.maximum(cols[i], cols[j])
        lo = jnp.minimum(cols[i], cols[j])
        cols[i] = hi
        cols[j] = lo
    for p in range(k - 1):
        m = jnp.max(cols[0], axis=1, keepdims=True)
        msk = cols[0] >= m
        for j in range(k - 1 - p):
            cols[j] = jnp.where(msk, cols[j + 1], cols[j])
    thr = jnp.max(cols[0], axis=1, keepdims=True)

    g = g_ref[...]
    tz = jnp.tanh(gamma * z)
    gate = 1.0 + jnp.where(tz > 0.0, beta_up, beta_dn) * tz
    gate = jnp.clip(gate, 0.05, 8.0)
    gate = jnp.where(az >= thr, gate, 1.0)

    dn = (((1,), (1,)), ((), ()))
    cs = jax.lax.dot_general(g, ema_n, dn, preferred_element_type=jnp.float32)
    gn2 = jax.lax.dot_general(g * g, jnp.ones((1, d), jnp.float32), dn,
                              preferred_element_type=jnp.float32)
    gn = jnp.sqrt(gn2)
    inv_gn = 1.0 / jnp.maximum(gn, 1e-12)
    cos = jnp.clip(cs * inv_gn, -1.0, 1.0)
    gate_cos = jnp.exp(-tau * cos)

    o_ref[...] = g * gate * gate_cos


def kernel(x, logit_decay, log_tau, log_beta_up, log_beta_dn, log_gamma):
    del logit_decay
    B, T, D = x.shape
    n = B * T
    k = min(_K, D)
    xf = x.reshape(n, D)

    params = jnp.stack([
        jnp.exp(log_tau),
        jax.nn.softplus(log_beta_up),
        jax.nn.softplus(log_beta_dn),
        jax.nn.softplus(log_gamma),
    ]).astype(jnp.float32)

    tb = 512
    grid = n // tb

    stats, gout = pl.pallas_call(
        _stats_kernel,
        grid=(grid,),
        in_specs=[pl.BlockSpec((tb, D), lambda i: (i, 0))],
        out_specs=[pl.BlockSpec((3, D), lambda i: (0, 0)),
                   pl.BlockSpec((tb, D), lambda i: (i, 0))],
        out_shape=[jax.ShapeDtypeStruct((3, D), jnp.float32),
                   jax.ShapeDtypeStruct((n, D), jnp.float32)],
    )(xf)

    out = pl.pallas_call(
        functools.partial(_main_kernel, n_tokens=float(n), k=k),
        grid=(grid,),
        in_specs=[
            pl.BlockSpec((tb, D), lambda i: (i, 0)),
            pl.BlockSpec((tb, D), lambda i: (i, 0)),
            pl.BlockSpec((3, D), lambda i: (0, 0)),
            pl.BlockSpec(memory_space=pltpu.SMEM),
        ],
        out_specs=pl.BlockSpec((tb, D), lambda i: (i, 0)),
        out_shape=jax.ShapeDtypeStruct((n, D), jnp.float32),
    )(xf, gout, stats, params)

    return out.reshape(B, T, D)
